# Initial kernel scaffold; baseline (speedup 1.0000x reference)
import jax
import jax.numpy as jnp
from jax.experimental import pallas as pl


def kernel(x, edge_index, conv1_W, conv1_b, pool1_W, pool1_b, conv2_W, conv2_b, pool2_W, pool2_b, conv3_W, conv3_b, pool3_W, pool3_b, lin1_W, lin1_b, lin2_W, lin2_b, lin3_W, lin3_b):
    raise NotImplementedError("write your pallas kernel here")


# speedup vs baseline: 1.0268x; 1.0268x over previous
import functools
import math

import jax
import jax.numpy as jnp
from jax import lax
from jax.experimental import pallas as pl
from jax.experimental.pallas import tpu as pltpu


def _mm_body(x_ref, w_ref, o_ref):
    o_ref[...] = jnp.dot(x_ref[...], w_ref[...],
                         preferred_element_type=jnp.float32)


def _matmul(x, W, BM=512):
    n, K = x.shape
    F = W.shape[1]
    npad = ((n + BM - 1) // BM) * BM
    if npad != n:
        x = jnp.pad(x, ((0, npad - n), (0, 0)))
    out = pl.pallas_call(
        _mm_body,
        grid=(npad // BM,),
        in_specs=[
            pl.BlockSpec((BM, K), lambda i: (i, 0)),
            pl.BlockSpec((K, F), lambda i: (0, 0)),
        ],
        out_specs=pl.BlockSpec((BM, F), lambda i: (i, 0)),
        out_shape=jax.ShapeDtypeStruct((npad, F), jnp.float32),
    )(x, W)
    return out[:n]


def _gcn(x, W, b, row, col, emask, n):
    h = _matmul(x, W)
    deg = jnp.zeros((n,), x.dtype).at[row].add(emask) + 1.0
    dinv = 1.0 / jnp.sqrt(deg)
    norm = dinv[row] * dinv[col] * emask
    agg = jnp.zeros((n, W.shape[1]), x.dtype).at[row].add(norm[:, None] * h[col])
    agg = agg + (dinv * dinv)[:, None] * h
    return agg + b


def _sagpool(x, row, col, emask, n, Ws, bs, ratio):
    score = _gcn(x, Ws, bs, row, col, emask, n)[:, 0]
    k = int(math.ceil(ratio * n))
    _, perm = jax.lax.top_k(score, k)
    x2 = x[perm] * jnp.tanh(score[perm])[:, None]
    mapping = jnp.full((n,), -1, dtype=jnp.int32).at[perm].set(
        jnp.arange(k, dtype=jnp.int32))
    r2 = mapping[row]
    c2 = mapping[col]
    valid = (r2 >= 0) & (c2 >= 0)
    emask2 = emask * valid.astype(x.dtype)
    r2 = jnp.where(valid, r2, 0)
    c2 = jnp.where(valid, c2, 0)
    return x2, r2, c2, emask2, k


def _readout(x):
    return jnp.concatenate(
        [jnp.mean(x, axis=0, keepdims=True),
         jnp.max(x, axis=0, keepdims=True)], axis=1)


def kernel(x, edge_index, conv1_W, conv1_b, pool1_W, pool1_b, conv2_W,
           conv2_b, pool2_W, pool2_b, conv3_W, conv3_b, pool3_W, pool3_b,
           lin1_W, lin1_b, lin2_W, lin2_b, lin3_W, lin3_b):
    row = edge_index[0]
    col = edge_index[1]
    n = x.shape[0]
    emask = jnp.ones((row.shape[0],), x.dtype)

    h = jax.nn.leaky_relu(_gcn(x, conv1_W, conv1_b, row, col, emask, n))
    h, row, col, emask, n = _sagpool(h, row, col, emask, n, pool1_W, pool1_b, 0.5)
    o1 = _readout(h)

    h = jax.nn.leaky_relu(_gcn(h, conv2_W, conv2_b, row, col, emask, n))
    h, row, col, emask, n = _sagpool(h, row, col, emask, n, pool2_W, pool2_b, 0.5)
    o2 = _readout(h)

    h = jax.nn.leaky_relu(_gcn(h, conv3_W, conv3_b, row, col, emask, n))
    h, row, col, emask, n = _sagpool(h, row, col, emask, n, pool3_W, pool3_b, 0.5)
    o3 = _readout(h)

    z = o1 + o2 + o3
    z = jax.nn.relu(z @ lin1_W + lin1_b)
    z = jax.nn.relu(z @ lin2_W + lin2_b)
    z = jax.nn.log_softmax(z @ lin3_W + lin3_b, axis=-1)
    return z


# speedup vs baseline: 1.1455x; 1.1156x over previous
import functools
import math

import jax
import jax.numpy as jnp
from jax import lax
from jax.experimental import pallas as pl
from jax.experimental.pallas import tpu as pltpu
from jax.experimental.pallas import tpu_sc as plsc

_NC = 2
_NS = 16
_NW = _NC * _NS
_CHUNK = 80
_W = 128


def _mm_body(x_ref, w_ref, o_ref):
    o_ref[...] = jnp.dot(x_ref[...], w_ref[...],
                         preferred_element_type=jnp.float32)


def _matmul(x, W, BM=512):
    n, K = x.shape
    F = W.shape[1]
    npad = ((n + BM - 1) // BM) * BM
    if npad != n:
        x = jnp.pad(x, ((0, npad - n), (0, 0)))
    out = pl.pallas_call(
        _mm_body,
        grid=(npad // BM,),
        in_specs=[
            pl.BlockSpec((BM, K), lambda i: (i, 0)),
            pl.BlockSpec((K, F), lambda i: (0, 0)),
        ],
        out_specs=pl.BlockSpec((BM, F), lambda i: (i, 0)),
        out_shape=jax.ShapeDtypeStruct((npad, F), jnp.float32),
    )(x, W)
    return out[:n]


def _gs_body(ept, npad, table_ref, col_ref, row_ref, out_ref,
             col_v, row_v, rows_v, zbuf, shared, sem):
    cid = lax.axis_index("c")
    sid = lax.axis_index("s")
    zero = jnp.zeros((16,), jnp.float32)
    for r in range(8):
        for j in range(_W // 16):
            zbuf[r, pl.ds(j * 16, 16)] = zero

    rpt = npad // _NS

    def zloop(i, c):
        pltpu.sync_copy(zbuf, shared.at[pl.ds(sid * rpt + i * 8, 8)])
        return c
    lax.fori_loop(0, rpt // 8, zloop, 0)
    plsc.subcore_barrier()

    base0 = (cid * _NS + sid) * ept

    def eloop(i, c):
        b = base0 + i * _CHUNK
        pltpu.sync_copy(col_ref.at[pl.ds(b, _CHUNK)], col_v)
        pltpu.sync_copy(row_ref.at[pl.ds(b, _CHUNK)], row_v)
        pltpu.async_copy(table_ref.at[col_v], rows_v, sem).wait()
        pltpu.sync_copy(rows_v, shared.at[row_v], add=True)
        return c
    lax.fori_loop(0, ept // _CHUNK, eloop, 0)
    plsc.subcore_barrier()
    pltpu.sync_copy(shared.at[pl.ds(sid * rpt, rpt)],
                    out_ref.at[cid, pl.ds(sid * rpt, rpt)])


def _gather_scatter_add(table, col, row, npad):
    T = table.shape[0]
    E = col.shape[0]
    quant = _NW * _CHUNK
    Epad = ((E + quant - 1) // quant) * quant
    if Epad != E:
        col = jnp.pad(col, (0, Epad - E), constant_values=T - 1)
        row = jnp.pad(row, (0, Epad - E), constant_values=npad - 1)
    ept = Epad // _NW
    mesh = plsc.VectorSubcoreMesh(core_axis_name="c", subcore_axis_name="s")
    f = pl.kernel(
        functools.partial(_gs_body, ept, npad),
        out_type=jax.ShapeDtypeStruct((_NC, npad, _W), jnp.float32),
        mesh=mesh,
        scratch_types=[
            pltpu.VMEM((_CHUNK,), jnp.int32),
            pltpu.VMEM((_CHUNK,), jnp.int32),
            pltpu.VMEM((_CHUNK, _W), jnp.float32),
            pltpu.VMEM((8, _W), jnp.float32),
            pltpu.VMEM_SHARED((npad, _W), jnp.float32),
            pltpu.SemaphoreType.DMA,
        ],
    )
    p = f(table, col, row)
    return p[0] + p[1]


def _readout(x):
    return jnp.concatenate(
        [jnp.mean(x, axis=0, keepdims=True),
         jnp.max(x, axis=0, keepdims=True)], axis=1)


def kernel(x, edge_index, conv1_W, conv1_b, pool1_W, pool1_b, conv2_W,
           conv2_b, pool2_W, pool2_b, conv3_W, conv3_b, pool3_W, pool3_b,
           lin1_W, lin1_b, lin2_W, lin2_b, lin3_W, lin3_b):
    row = edge_index[0]
    col = edge_index[1]
    h = x
    outs = []
    for Wc, bc, Ws, bs in ((conv1_W, conv1_b, pool1_W, pool1_b),
                           (conv2_W, conv2_b, pool2_W, pool2_b),
                           (conv3_W, conv3_b, pool3_W, pool3_b)):
        n = h.shape[0]
        npad = ((n + 1 + 127) // 128) * 128

        validc = (col < n).astype(jnp.float32)
        deg = jnp.zeros((n + 1,), jnp.float32).at[row].add(validc)[:n] + 1.0
        dinv = 1.0 / jnp.sqrt(deg)
        dinvx = jnp.concatenate([dinv, jnp.zeros((1,), jnp.float32)])

        hm = _matmul(h, Wc)
        table = jnp.zeros((n + 1, _W), jnp.float32).at[:n].set(
            dinv[:, None] * hm)
        part = _gather_scatter_add(table, col, row, npad)[:n]
        agg = dinv[:, None] * part + (dinv * dinv)[:, None] * hm + bc
        hc = jax.nn.leaky_relu(agg)

        sm = _matmul(hc, jnp.pad(Ws, ((0, 0), (0, 127))))[:, 0]
        smx = jnp.concatenate([sm, jnp.zeros((1,), jnp.float32)])
        val = dinvx[col] * smx[col]
        sp = jnp.zeros((n + 1,), jnp.float32).at[row].add(val)[:n]
        score = dinv * sp + dinv * dinv * sm + bs[0]

        k = int(math.ceil(0.5 * n))
        _, perm = lax.top_k(score, k)
        h = hc[perm] * jnp.tanh(score[perm])[:, None]
        mapping = jnp.full((n + 1,), k, jnp.int32).at[perm].set(
            jnp.arange(k, dtype=jnp.int32))
        row = mapping[row]
        col = mapping[col]
        outs.append(_readout(h))

    z = outs[0] + outs[1] + outs[2]
    z = jax.nn.relu(z @ lin1_W + lin1_b)
    z = jax.nn.relu(z @ lin2_W + lin2_b)
    z = jax.nn.log_softmax(z @ lin3_W + lin3_b, axis=-1)
    return z


# speedup vs baseline: 1.7345x; 1.5142x over previous
import functools
import math

import jax
import jax.numpy as jnp
from jax import lax
from jax.experimental import pallas as pl
from jax.experimental.pallas import tpu as pltpu
from jax.experimental.pallas import tpu_sc as plsc

_NC = 2
_NS = 16
_NW = _NC * _NS
_CHUNK = 80
_W = 128
_G = 1024


def _mm_body(x_ref, w_ref, o_ref):
    o_ref[...] = jnp.dot(x_ref[...], w_ref[...],
                         preferred_element_type=jnp.float32)


def _matmul(x, W, BM=512):
    n, K = x.shape
    F = W.shape[1]
    npad = ((n + BM - 1) // BM) * BM
    if npad != n:
        x = jnp.pad(x, ((0, npad - n), (0, 0)))
    out = pl.pallas_call(
        _mm_body,
        grid=(npad // BM,),
        in_specs=[
            pl.BlockSpec((BM, K), lambda i: (i, 0)),
            pl.BlockSpec((K, F), lambda i: (0, 0)),
        ],
        out_specs=pl.BlockSpec((BM, F), lambda i: (i, 0)),
        out_shape=jax.ShapeDtypeStruct((npad, F), jnp.float32),
    )(x, W)
    return out[:n]


def _gs_body(ept, npad, table_ref, col_ref, row_ref, out_ref,
             col_v, row_v, rows_v, zbuf, shared, sem):
    cid = lax.axis_index("c")
    sid = lax.axis_index("s")
    zero = jnp.zeros((16,), jnp.float32)
    for r in range(8):
        for j in range(_W // 16):
            zbuf[r, pl.ds(j * 16, 16)] = zero

    rpt = npad // _NS

    def zloop(i, c):
        pltpu.sync_copy(zbuf, shared.at[pl.ds(sid * rpt + i * 8, 8)])
        return c
    lax.fori_loop(0, rpt // 8, zloop, 0)
    plsc.subcore_barrier()

    base0 = (cid * _NS + sid) * ept

    def eloop(i, c):
        b = base0 + i * _CHUNK
        pltpu.sync_copy(col_ref.at[pl.ds(b, _CHUNK)], col_v)
        pltpu.sync_copy(row_ref.at[pl.ds(b, _CHUNK)], row_v)
        pltpu.async_copy(table_ref.at[col_v], rows_v, sem).wait()
        pltpu.sync_copy(rows_v, shared.at[row_v], add=True)
        return c
    lax.fori_loop(0, ept // _CHUNK, eloop, 0)
    plsc.subcore_barrier()
    pltpu.sync_copy(shared.at[pl.ds(sid * rpt, rpt)],
                    out_ref.at[cid, pl.ds(sid * rpt, rpt)])


def _gather_scatter_add(table, col, row, npad):
    T = table.shape[0]
    E = col.shape[0]
    quant = _NW * _CHUNK
    Epad = ((E + quant - 1) // quant) * quant
    if Epad != E:
        col = jnp.pad(col, (0, Epad - E), constant_values=T - 1)
        row = jnp.pad(row, (0, Epad - E), constant_values=npad - 1)
    ept = Epad // _NW
    mesh = plsc.VectorSubcoreMesh(core_axis_name="c", subcore_axis_name="s")
    f = pl.kernel(
        functools.partial(_gs_body, ept, npad),
        out_type=jax.ShapeDtypeStruct((_NC, npad, _W), jnp.float32),
        mesh=mesh,
        scratch_types=[
            pltpu.VMEM((_CHUNK,), jnp.int32),
            pltpu.VMEM((_CHUNK,), jnp.int32),
            pltpu.VMEM((_CHUNK, _W), jnp.float32),
            pltpu.VMEM((8, _W), jnp.float32),
            pltpu.VMEM_SHARED((npad, _W), jnp.float32),
            pltpu.SemaphoreType.DMA,
        ],
    )
    p = f(table, col, row)
    return p[0] + p[1]


def _readout(x):
    return jnp.concatenate(
        [jnp.mean(x, axis=0, keepdims=True),
         jnp.max(x, axis=0, keepdims=True)], axis=1)


def kernel(x, edge_index, conv1_W, conv1_b, pool1_W, pool1_b, conv2_W,
           conv2_b, pool2_W, pool2_b, conv3_W, conv3_b, pool3_W, pool3_b,
           lin1_W, lin1_b, lin2_W, lin2_b, lin3_W, lin3_b):
    row = edge_index[0]
    col = edge_index[1]
    h = x
    gi = jnp.arange(row.shape[0], dtype=jnp.int32) & (_G - 1)
    outs = []
    for Wc, bc, Ws, bs in ((conv1_W, conv1_b, pool1_W, pool1_b),
                           (conv2_W, conv2_b, pool2_W, pool2_b),
                           (conv3_W, conv3_b, pool3_W, pool3_b)):
        n = h.shape[0]
        npad = ((n + _G + 127) // 128) * 128

        validc = (col < n).astype(jnp.float32)
        deg = jnp.zeros((n + 1,), jnp.float32).at[row].add(validc)[:n] + 1.0
        dinv = 1.0 / jnp.sqrt(deg)
        dinvx = jnp.concatenate([dinv, jnp.zeros((1,), jnp.float32)])

        hm = _matmul(h, Wc)
        table = jnp.zeros((n + _G, _W), jnp.float32).at[:n].set(
            dinv[:, None] * hm)
        rowk = jnp.where(row >= n, n + gi, row)
        colk = jnp.where(col >= n, n + gi, col)
        part = _gather_scatter_add(table, colk, rowk, npad)[:n]
        agg = dinv[:, None] * part + (dinv * dinv)[:, None] * hm + bc
        hc = jax.nn.leaky_relu(agg)

        sm = _matmul(hc, jnp.pad(Ws, ((0, 0), (0, 127))))[:, 0]
        smx = jnp.concatenate([sm, jnp.zeros((1,), jnp.float32)])
        val = dinvx[col] * smx[col]
        sp = jnp.zeros((n + 1,), jnp.float32).at[row].add(val)[:n]
        score = dinv * sp + dinv * dinv * sm + bs[0]

        k = int(math.ceil(0.5 * n))
        _, perm = lax.top_k(score, k)
        h = hc[perm] * jnp.tanh(score[perm])[:, None]
        mapping = jnp.full((n + 1,), k, jnp.int32).at[perm].set(
            jnp.arange(k, dtype=jnp.int32))
        row = mapping[row]
        col = mapping[col]
        outs.append(_readout(h))

    z = outs[0] + outs[1] + outs[2]
    z = jax.nn.relu(z @ lin1_W + lin1_b)
    z = jax.nn.relu(z @ lin2_W + lin2_b)
    z = jax.nn.log_softmax(z @ lin3_W + lin3_b, axis=-1)
    return z
